# Initial kernel scaffold; baseline (speedup 1.0000x reference)
import jax
import jax.numpy as jnp
from jax.experimental import pallas as pl


def kernel(x, edge_index, batch, W_in, b_in, conv_W, conv_b, mu_W, mu_b, lv_W, lv_b, p1_W, p1_b, p2_W, p2_b):
    raise NotImplementedError("write your pallas kernel here")


# speedup vs baseline: 17.8379x; 17.8379x over previous
import functools

import jax
import jax.numpy as jnp
from jax import lax
from jax.experimental import pallas as pl
from jax.experimental.pallas import tpu as pltpu
from jax.experimental.pallas import tpu_sc as plsc

F32 = jnp.float32
HIGHEST = jax.lax.Precision.HIGHEST

N = 100000
D = 32
DH = 16
G = 512
CHUNK = 1024
SUB = 128
ROWS_PER_CHUNK = CHUNK // SUB
NTILES = 16
R = N + 32
BN = 4000
GRID = N // BN


def _dot(a, b):
    return jax.lax.dot_general(a, b, (((1,), (0,)), ((), ())),
                               precision=HIGHEST,
                               preferred_element_type=F32)


def _sc_agg_body(gflat, srcoff, dst2d, zeros_hbm, outflat,
                 table, srcv, dstv, rowsv, gsem, ssem):
    c = lax.axis_index("c")
    s = lax.axis_index("s")
    cpt = srcoff.shape[0] // (2 * NTILES * ROWS_PER_CHUNK)

    @pl.when(s < 8)
    def _zero():
        nr = R // 8
        pltpu.sync_copy(zeros_hbm.at[pl.ds(s * nr, nr)],
                        table.at[pl.ds(s * nr, nr)])
    plsc.subcore_barrier()

    def chunk(k, carry):
        rb_dst = s * (cpt * ROWS_PER_CHUNK) + k * ROWS_PER_CHUNK
        rb_src = c * (srcoff.shape[0] // 2) + rb_dst
        pltpu.sync_copy(srcoff.at[pl.ds(rb_src, ROWS_PER_CHUNK)], srcv)
        pltpu.sync_copy(dst2d.at[pl.ds(rb_dst, ROWS_PER_CHUNK)], dstv)
        gets = [pltpu.async_copy(gflat.at[srcv.at[j]],
                                 rowsv.at[pl.ds(j * SUB, SUB)], gsem)
                for j in range(ROWS_PER_CHUNK)]
        for h in gets:
            h.wait()
        puts = [pltpu.async_copy(rowsv.at[pl.ds(j * SUB, SUB)],
                                 table.at[dstv.at[j]], ssem, add=True)
                for j in range(ROWS_PER_CHUNK)]
        for h in puts:
            h.wait()
        return carry

    lax.fori_loop(0, cpt, chunk, 0)
    plsc.subcore_barrier()

    @pl.when(s < 4)
    def _out():
        nr = N // 4
        pltpu.sync_copy(table.at[pl.ds(s * nr, nr)],
                        outflat.at[pl.ds(c * N + s * nr, nr)])


def _make_sc_agg(ep):
    return functools.partial(
        pl.kernel,
        out_type=jax.ShapeDtypeStruct((2 * N, DH), F32),
        mesh=plsc.VectorSubcoreMesh(core_axis_name="c", subcore_axis_name="s"),
        compiler_params=pltpu.CompilerParams(use_tc_tiling_on_sc=False),
        scratch_types=[
            pltpu.VMEM_SHARED((R, DH), F32),
            pltpu.VMEM((ROWS_PER_CHUNK, SUB), jnp.int32),
            pltpu.VMEM((ROWS_PER_CHUNK, SUB), jnp.int32),
            pltpu.VMEM((CHUNK, DH), F32),
            pltpu.SemaphoreType.DMA,
            pltpu.SemaphoreType.DMA,
        ],
    )(_sc_agg_body)


def _sc_deg_body(dst2d, zeros_hbm, ones_hbm, outflat,
                 table, dstv, onesv, ssem):
    c = lax.axis_index("c")
    s = lax.axis_index("s")
    cpt = dst2d.shape[0] // (2 * NTILES * ROWS_PER_CHUNK)

    @pl.when(s < 8)
    def _zero():
        nr = R // 8
        pltpu.sync_copy(zeros_hbm.at[pl.ds(s * nr, nr)],
                        table.at[pl.ds(s * nr, nr)])
    pltpu.sync_copy(ones_hbm, onesv)
    plsc.subcore_barrier()

    def chunk(k, carry):
        rb = (c * (dst2d.shape[0] // 2)
              + s * (cpt * ROWS_PER_CHUNK) + k * ROWS_PER_CHUNK)
        pltpu.sync_copy(dst2d.at[pl.ds(rb, ROWS_PER_CHUNK)], dstv)
        puts = [pltpu.async_copy(onesv, table.at[dstv.at[j]], ssem, add=True)
                for j in range(ROWS_PER_CHUNK)]
        for h in puts:
            h.wait()
        return carry

    lax.fori_loop(0, cpt, chunk, 0)
    plsc.subcore_barrier()

    @pl.when(s < 4)
    def _out():
        nr = N // 4
        pltpu.sync_copy(table.at[pl.ds(s * nr, nr)],
                        outflat.at[pl.ds(c * N + s * nr, nr)])


def _make_sc_deg():
    return functools.partial(
        pl.kernel,
        out_type=jax.ShapeDtypeStruct((2 * N, DH), F32),
        mesh=plsc.VectorSubcoreMesh(core_axis_name="c", subcore_axis_name="s"),
        compiler_params=pltpu.CompilerParams(use_tc_tiling_on_sc=False),
        scratch_types=[
            pltpu.VMEM_SHARED((R, DH), F32),
            pltpu.VMEM((ROWS_PER_CHUNK, SUB), jnp.int32),
            pltpu.VMEM((SUB, DH), F32),
            pltpu.SemaphoreType.DMA,
        ],
    )(_sc_deg_body)


def _prep_mm_body(x_ref, win_ref, bin_ref, w0_ref, b0_ref, deg_ref,
                  g_ref, self_ref, dis_ref, inv_ref):
    d = deg_ref[0, :, 0:1] + deg_ref[1, :, 0:1] + 1.0
    inv = 1.0 / d
    dis = jax.lax.rsqrt(d)
    h0 = _dot(x_ref[...], win_ref[...]) + bin_ref[...]
    hw = _dot(h0, w0_ref[...])
    g = hw * dis
    g_ref[0] = g[:, :DH]
    g_ref[1] = g[:, DH:]
    self_ref[...] = hw * inv + b0_ref[...]
    dis_ref[...] = dis
    inv_ref[...] = inv


def _mm_body(acc_ref, selfp_ref, dis_ref, inv_ref, w_ref, b_ref,
             g_ref, self_ref):
    acc = jnp.concatenate([acc_ref[0], acc_ref[1]], axis=1)
    h = dis_ref[...] * acc + selfp_ref[...]
    hw = _dot(h, w_ref[...])
    g = hw * dis_ref[...]
    g_ref[0] = g[:, :DH]
    g_ref[1] = g[:, DH:]
    self_ref[...] = hw * inv_ref[...] + b_ref[...]


def _pool_body(acc_ref, self_ref, dis_ref, batch_ref, hg_ref, cnt_ref):
    h4 = (dis_ref[...] * jnp.concatenate([acc_ref[0], acc_ref[1]], axis=1)
          + self_ref[...])
    onehot = (batch_ref[...] ==
              jax.lax.broadcasted_iota(jnp.int32, (BN, G), 1)).astype(F32)
    hg_blk = jax.lax.dot_general(onehot, h4, (((0,), (0,)), ((), ())),
                                 precision=HIGHEST,
                                 preferred_element_type=F32)
    cnt_blk = jax.lax.dot_general(onehot, jnp.ones((BN, 1), F32),
                                  (((0,), (0,)), ((), ())),
                                  precision=HIGHEST,
                                  preferred_element_type=F32)

    @pl.when(pl.program_id(0) == 0)
    def _init():
        hg_ref[...] = hg_blk
        cnt_ref[...] = cnt_blk

    @pl.when(pl.program_id(0) != 0)
    def _acc():
        hg_ref[...] += hg_blk
        cnt_ref[...] += cnt_blk


def _head_body(hg_ref, cnt_ref, muw_ref, mub_ref, lvw_ref, lvb_ref,
               p1w_ref, p1b_ref, p2w_ref, p2b_ref,
               mu_ref, lv_ref, prop_ref):
    hg = hg_ref[...] / jnp.maximum(cnt_ref[...], 1.0)
    mu = _dot(hg, muw_ref[...]) + mub_ref[...]
    lv = _dot(hg, lvw_ref[...]) + lvb_ref[...]
    hid = jnp.maximum(_dot(mu, p1w_ref[...]) + p1b_ref[...], 0.0)
    mu_ref[...] = mu
    lv_ref[...] = lv
    prop_ref[...] = _dot(hid, p2w_ref[...]) + p2b_ref[...]


def _full(shape):
    nd = len(shape)
    return pl.BlockSpec(shape, lambda *b: (0,) * nd)


def _rows(width):
    return pl.BlockSpec((BN, width), lambda b: (b, 0))


_SPEC_G = pl.BlockSpec((2, BN, DH), lambda b: (0, b, 0))


def kernel(x, edge_index, batch, W_in, b_in, conv_W, conv_b,
           mu_W, mu_b, lv_W, lv_b, p1_W, p1_b, p2_W, p2_b):
    n, df = x.shape
    e = edge_index.shape[1]
    assert n == N and df == 128 and conv_W.shape == (4, D, D)

    ep = ((e + 2 * NTILES * CHUNK - 1) // (2 * NTILES * CHUNK)) * (2 * NTILES * CHUNK)
    src = jnp.concatenate(
        [edge_index[0], jnp.zeros((ep - e,), jnp.int32)]).reshape(-1, SUB)
    dst = jnp.concatenate(
        [edge_index[1], jnp.full((ep - e,), N, jnp.int32)]).reshape(-1, SUB)
    srcoff = jnp.concatenate([src, src + N], axis=0)

    zeros_hbm = jnp.zeros((R, DH), F32)
    ones_hbm = jnp.ones((SUB, DH), F32)

    deg2 = _make_sc_deg()(dst, zeros_hbm, ones_hbm).reshape(2, N, DH)

    sc_agg = _make_sc_agg(ep)

    b_in2 = b_in.reshape(1, D)
    conv_b2 = conv_b.reshape(4, 1, D)

    g, self_t, dis, inv = pl.pallas_call(
        _prep_mm_body,
        grid=(GRID,),
        in_specs=[_rows(df), _full((df, D)), _full((1, D)),
                  _full((D, D)), _full((1, D)),
                  pl.BlockSpec((2, BN, DH), lambda b: (0, b, 0))],
        out_specs=[_SPEC_G, _rows(D), _rows(1), _rows(1)],
        out_shape=[jax.ShapeDtypeStruct((2, N, DH), F32),
                   jax.ShapeDtypeStruct((N, D), F32),
                   jax.ShapeDtypeStruct((N, 1), F32),
                   jax.ShapeDtypeStruct((N, 1), F32)],
    )(x, W_in, b_in2, conv_W[0], conv_b2[0], deg2)

    for i in range(1, 4):
        acc = sc_agg(g.reshape(2 * N, DH), srcoff, dst,
                     zeros_hbm).reshape(2, N, DH)
        g, self_t = pl.pallas_call(
            _mm_body,
            grid=(GRID,),
            in_specs=[_SPEC_G, _rows(D), _rows(1), _rows(1),
                      _full((D, D)), _full((1, D))],
            out_specs=[_SPEC_G, _rows(D)],
            out_shape=[jax.ShapeDtypeStruct((2, N, DH), F32),
                       jax.ShapeDtypeStruct((N, D), F32)],
        )(acc, self_t, dis, inv, conv_W[i], conv_b2[i])

    acc4 = sc_agg(g.reshape(2 * N, DH), srcoff, dst,
                  zeros_hbm).reshape(2, N, DH)

    hg_sum, cnt = pl.pallas_call(
        _pool_body,
        grid=(GRID,),
        in_specs=[_SPEC_G, _rows(D), _rows(1), _rows(1)],
        out_specs=[_full((G, D)), _full((G, 1))],
        out_shape=[jax.ShapeDtypeStruct((G, D), F32),
                   jax.ShapeDtypeStruct((G, 1), F32)],
    )(acc4, self_t, dis, batch.reshape(N, 1))

    mu, lv, prop = pl.pallas_call(
        _head_body,
        in_specs=[_full((G, D)), _full((G, 1)),
                  _full((D, 32)), _full((1, 32)),
                  _full((D, 32)), _full((1, 32)),
                  _full((32, D)), _full((1, D)),
                  _full((D, 1)), _full((1, 1))],
        out_specs=[_full((G, 32)), _full((G, 32)), _full((G, 1))],
        out_shape=[jax.ShapeDtypeStruct((G, 32), F32),
                   jax.ShapeDtypeStruct((G, 32), F32),
                   jax.ShapeDtypeStruct((G, 1), F32)],
    )(hg_sum, cnt, mu_W, mu_b.reshape(1, 32), lv_W, lv_b.reshape(1, 32),
      p1_W, p1_b.reshape(1, D), p2_W, p2_b.reshape(1, 1))

    return (mu, lv, prop)


# speedup vs baseline: 19.5861x; 1.0980x over previous
import functools

import jax
import jax.numpy as jnp
from jax import lax
from jax.experimental import pallas as pl
from jax.experimental.pallas import tpu as pltpu
from jax.experimental.pallas import tpu_sc as plsc

F32 = jnp.float32
HIGHEST = jax.lax.Precision.HIGHEST

N = 100000
D = 32
DH = 16
G = 512
CHUNK = 1024
SUB = 128
ROWS_PER_CHUNK = CHUNK // SUB
AGG_CHUNK = 512
AGG_RPC = AGG_CHUNK // SUB
NTILES = 16
R = N + 32
BN = 4000
GRID = N // BN


def _dot(a, b):
    return jax.lax.dot_general(a, b, (((1,), (0,)), ((), ())),
                               precision=HIGHEST,
                               preferred_element_type=F32)


def _sc_agg_body(gflat, srcoff, dst2d, zeros_hbm, outflat,
                 table, srcv, dstv, rowsv, gsem, ssem):
    c = lax.axis_index("c")
    s = lax.axis_index("s")
    rpc = AGG_RPC
    cpt = srcoff.shape[0] // (2 * NTILES * rpc)
    half = srcoff.shape[0] // 2

    @pl.when(s < 8)
    def _zero():
        nr = R // 8
        pltpu.sync_copy(zeros_hbm.at[pl.ds(s * nr, nr)],
                        table.at[pl.ds(s * nr, nr)])
    plsc.subcore_barrier()

    def load_idx(buf, k):
        rb_dst = s * (cpt * rpc) + k * rpc
        pltpu.sync_copy(srcoff.at[pl.ds(c * half + rb_dst, rpc)],
                        srcv.at[buf])
        pltpu.sync_copy(dst2d.at[pl.ds(rb_dst, rpc)], dstv.at[buf])

    def fire_gathers(buf):
        for j in range(rpc):
            pltpu.async_copy(gflat.at[srcv.at[buf, j]],
                             rowsv.at[buf, pl.ds(j * SUB, SUB)], gsem)

    def drain_gathers(buf):
        for j in range(rpc):
            pltpu.make_async_copy(gflat.at[srcv.at[buf, j]],
                                  rowsv.at[buf, pl.ds(j * SUB, SUB)],
                                  gsem).wait()

    def scatters(buf):
        puts = [pltpu.async_copy(rowsv.at[buf, pl.ds(j * SUB, SUB)],
                                 table.at[dstv.at[buf, j]], ssem, add=True)
                for j in range(rpc)]
        for h in puts:
            h.wait()

    load_idx(0, 0)
    fire_gathers(0)

    def pair(kk, carry):
        for b in range(2):
            k = 2 * kk + b
            nb = 1 - b
            if b == 0:
                load_idx(nb, k + 1)
                drain_gathers(b)
                fire_gathers(nb)
            else:
                @pl.when(kk < cpt // 2 - 1)
                def _pre():
                    load_idx(nb, k + 1)
                drain_gathers(b)

                @pl.when(kk < cpt // 2 - 1)
                def _fire():
                    fire_gathers(nb)
            scatters(b)
        return carry

    lax.fori_loop(0, cpt // 2, pair, 0)
    plsc.subcore_barrier()

    @pl.when(s < 4)
    def _out():
        nr = N // 4
        pltpu.sync_copy(table.at[pl.ds(s * nr, nr)],
                        outflat.at[pl.ds(c * N + s * nr, nr)])


def _make_sc_agg(ep):
    return functools.partial(
        pl.kernel,
        out_type=jax.ShapeDtypeStruct((2 * N, DH), F32),
        mesh=plsc.VectorSubcoreMesh(core_axis_name="c", subcore_axis_name="s"),
        compiler_params=pltpu.CompilerParams(use_tc_tiling_on_sc=False),
        scratch_types=[
            pltpu.VMEM_SHARED((R, DH), F32),
            pltpu.VMEM((2, AGG_RPC, SUB), jnp.int32),
            pltpu.VMEM((2, AGG_RPC, SUB), jnp.int32),
            pltpu.VMEM((2, AGG_CHUNK, DH), F32),
            pltpu.SemaphoreType.DMA,
            pltpu.SemaphoreType.DMA,
        ],
    )(_sc_agg_body)


def _sc_deg_body(dst2d, zeros_hbm, ones_hbm, outflat,
                 table, dstv, onesv, ssem):
    c = lax.axis_index("c")
    s = lax.axis_index("s")
    cpt = dst2d.shape[0] // (2 * NTILES * ROWS_PER_CHUNK)

    @pl.when(s < 8)
    def _zero():
        nr = R // 8
        pltpu.sync_copy(zeros_hbm.at[pl.ds(s * nr, nr)],
                        table.at[pl.ds(s * nr, nr)])
    pltpu.sync_copy(ones_hbm, onesv)
    plsc.subcore_barrier()

    def chunk(k, carry):
        rb = (c * (dst2d.shape[0] // 2)
              + s * (cpt * ROWS_PER_CHUNK) + k * ROWS_PER_CHUNK)
        pltpu.sync_copy(dst2d.at[pl.ds(rb, ROWS_PER_CHUNK)], dstv)
        puts = [pltpu.async_copy(onesv, table.at[dstv.at[j]], ssem, add=True)
                for j in range(ROWS_PER_CHUNK)]
        for h in puts:
            h.wait()
        return carry

    lax.fori_loop(0, cpt, chunk, 0)
    plsc.subcore_barrier()

    @pl.when(s < 4)
    def _out():
        nr = N // 4
        pltpu.sync_copy(table.at[pl.ds(s * nr, nr)],
                        outflat.at[pl.ds(c * N + s * nr, nr)])


def _make_sc_deg():
    return functools.partial(
        pl.kernel,
        out_type=jax.ShapeDtypeStruct((2 * N, DH), F32),
        mesh=plsc.VectorSubcoreMesh(core_axis_name="c", subcore_axis_name="s"),
        compiler_params=pltpu.CompilerParams(use_tc_tiling_on_sc=False),
        scratch_types=[
            pltpu.VMEM_SHARED((R, DH), F32),
            pltpu.VMEM((ROWS_PER_CHUNK, SUB), jnp.int32),
            pltpu.VMEM((SUB, DH), F32),
            pltpu.SemaphoreType.DMA,
        ],
    )(_sc_deg_body)


def _prep_mm_body(x_ref, win_ref, bin_ref, w0_ref, b0_ref, deg_ref,
                  g_ref, self_ref, dis_ref, inv_ref):
    d = deg_ref[0, :, 0:1] + deg_ref[1, :, 0:1] + 1.0
    inv = 1.0 / d
    dis = jax.lax.rsqrt(d)
    h0 = _dot(x_ref[...], win_ref[...]) + bin_ref[...]
    hw = _dot(h0, w0_ref[...])
    g = hw * dis
    g_ref[0] = g[:, :DH]
    g_ref[1] = g[:, DH:]
    self_ref[...] = hw * inv + b0_ref[...]
    dis_ref[...] = dis
    inv_ref[...] = inv


def _mm_body(acc_ref, selfp_ref, dis_ref, inv_ref, w_ref, b_ref,
             g_ref, self_ref):
    acc = jnp.concatenate([acc_ref[0], acc_ref[1]], axis=1)
    h = dis_ref[...] * acc + selfp_ref[...]
    hw = _dot(h, w_ref[...])
    g = hw * dis_ref[...]
    g_ref[0] = g[:, :DH]
    g_ref[1] = g[:, DH:]
    self_ref[...] = hw * inv_ref[...] + b_ref[...]


def _pool_body(acc_ref, self_ref, dis_ref, batch_ref, hg_ref, cnt_ref):
    h4 = (dis_ref[...] * jnp.concatenate([acc_ref[0], acc_ref[1]], axis=1)
          + self_ref[...])
    onehot = (batch_ref[...] ==
              jax.lax.broadcasted_iota(jnp.int32, (BN, G), 1)).astype(F32)
    hg_blk = jax.lax.dot_general(onehot, h4, (((0,), (0,)), ((), ())),
                                 precision=HIGHEST,
                                 preferred_element_type=F32)
    cnt_blk = jax.lax.dot_general(onehot, jnp.ones((BN, 1), F32),
                                  (((0,), (0,)), ((), ())),
                                  precision=HIGHEST,
                                  preferred_element_type=F32)

    @pl.when(pl.program_id(0) == 0)
    def _init():
        hg_ref[...] = hg_blk
        cnt_ref[...] = cnt_blk

    @pl.when(pl.program_id(0) != 0)
    def _acc():
        hg_ref[...] += hg_blk
        cnt_ref[...] += cnt_blk


def _head_body(hg_ref, cnt_ref, muw_ref, mub_ref, lvw_ref, lvb_ref,
               p1w_ref, p1b_ref, p2w_ref, p2b_ref,
               mu_ref, lv_ref, prop_ref):
    hg = hg_ref[...] / jnp.maximum(cnt_ref[...], 1.0)
    mu = _dot(hg, muw_ref[...]) + mub_ref[...]
    lv = _dot(hg, lvw_ref[...]) + lvb_ref[...]
    hid = jnp.maximum(_dot(mu, p1w_ref[...]) + p1b_ref[...], 0.0)
    mu_ref[...] = mu
    lv_ref[...] = lv
    prop_ref[...] = _dot(hid, p2w_ref[...]) + p2b_ref[...]


def _full(shape):
    nd = len(shape)
    return pl.BlockSpec(shape, lambda *b: (0,) * nd)


def _rows(width):
    return pl.BlockSpec((BN, width), lambda b: (b, 0))


_SPEC_G = pl.BlockSpec((2, BN, DH), lambda b: (0, b, 0))


def kernel(x, edge_index, batch, W_in, b_in, conv_W, conv_b,
           mu_W, mu_b, lv_W, lv_b, p1_W, p1_b, p2_W, p2_b):
    n, df = x.shape
    e = edge_index.shape[1]
    assert n == N and df == 128 and conv_W.shape == (4, D, D)

    ep = ((e + 2 * NTILES * CHUNK - 1) // (2 * NTILES * CHUNK)) * (2 * NTILES * CHUNK)
    src = jnp.concatenate(
        [edge_index[0], jnp.zeros((ep - e,), jnp.int32)]).reshape(-1, SUB)
    dst = jnp.concatenate(
        [edge_index[1], jnp.full((ep - e,), N, jnp.int32)]).reshape(-1, SUB)
    srcoff = jnp.concatenate([src, src + N], axis=0)

    zeros_hbm = jnp.zeros((R, DH), F32)
    ones_hbm = jnp.ones((SUB, DH), F32)

    deg2 = _make_sc_deg()(dst, zeros_hbm, ones_hbm).reshape(2, N, DH)

    sc_agg = _make_sc_agg(ep)

    b_in2 = b_in.reshape(1, D)
    conv_b2 = conv_b.reshape(4, 1, D)

    g, self_t, dis, inv = pl.pallas_call(
        _prep_mm_body,
        grid=(GRID,),
        in_specs=[_rows(df), _full((df, D)), _full((1, D)),
                  _full((D, D)), _full((1, D)),
                  pl.BlockSpec((2, BN, DH), lambda b: (0, b, 0))],
        out_specs=[_SPEC_G, _rows(D), _rows(1), _rows(1)],
        out_shape=[jax.ShapeDtypeStruct((2, N, DH), F32),
                   jax.ShapeDtypeStruct((N, D), F32),
                   jax.ShapeDtypeStruct((N, 1), F32),
                   jax.ShapeDtypeStruct((N, 1), F32)],
    )(x, W_in, b_in2, conv_W[0], conv_b2[0], deg2)

    for i in range(1, 4):
        acc = sc_agg(g.reshape(2 * N, DH), srcoff, dst,
                     zeros_hbm).reshape(2, N, DH)
        g, self_t = pl.pallas_call(
            _mm_body,
            grid=(GRID,),
            in_specs=[_SPEC_G, _rows(D), _rows(1), _rows(1),
                      _full((D, D)), _full((1, D))],
            out_specs=[_SPEC_G, _rows(D)],
            out_shape=[jax.ShapeDtypeStruct((2, N, DH), F32),
                       jax.ShapeDtypeStruct((N, D), F32)],
        )(acc, self_t, dis, inv, conv_W[i], conv_b2[i])

    acc4 = sc_agg(g.reshape(2 * N, DH), srcoff, dst,
                  zeros_hbm).reshape(2, N, DH)

    hg_sum, cnt = pl.pallas_call(
        _pool_body,
        grid=(GRID,),
        in_specs=[_SPEC_G, _rows(D), _rows(1), _rows(1)],
        out_specs=[_full((G, D)), _full((G, 1))],
        out_shape=[jax.ShapeDtypeStruct((G, D), F32),
                   jax.ShapeDtypeStruct((G, 1), F32)],
    )(acc4, self_t, dis, batch.reshape(N, 1))

    mu, lv, prop = pl.pallas_call(
        _head_body,
        in_specs=[_full((G, D)), _full((G, 1)),
                  _full((D, 32)), _full((1, 32)),
                  _full((D, 32)), _full((1, 32)),
                  _full((32, D)), _full((1, D)),
                  _full((D, 1)), _full((1, 1))],
        out_specs=[_full((G, 32)), _full((G, 32)), _full((G, 1))],
        out_shape=[jax.ShapeDtypeStruct((G, 32), F32),
                   jax.ShapeDtypeStruct((G, 32), F32),
                   jax.ShapeDtypeStruct((G, 1), F32)],
    )(hg_sum, cnt, mu_W, mu_b.reshape(1, 32), lv_W, lv_b.reshape(1, 32),
      p1_W, p1_b.reshape(1, D), p2_W, p2_b.reshape(1, 1))

    return (mu, lv, prop)
